# baseline (speedup 1.0000x reference)
import functools

import jax
import jax.numpy as jnp
from jax import lax
from jax.experimental import pallas as pl
from jax.experimental.pallas import tpu as pltpu
from jax.experimental.pallas import tpu_sc as plsc

IOU_THR = 0.4
BI = 512
BJ = 512
BIG = 1e30
NB = 64
NT = 16
PW = 128


def _bucket_body(pk_hbm, bu_hbm, perm_hbm, bstart_hbm, hists_hbm,
                 rowsv, buv, posv, histv, allh, basev, bigv,
                 hist_s, cnt_s, sem):
    wid = lax.axis_index("s")
    npad = bu_hbm.shape[0]
    m = npad // NT
    base = wid * m
    pltpu.sync_copy(pk_hbm.at[pl.ds(base, m)], rowsv)
    pltpu.sync_copy(bu_hbm.at[pl.ds(base, m)], buv)

    lanes = lax.iota(jnp.int32, 16)

    def bigloop(r, carry):
        for c in range(PW // 16):
            bigv[r, pl.ds(c * 16, 16)] = jnp.full((16,), BIG, jnp.float32)
        return carry

    lax.fori_loop(0, 128, bigloop, jnp.int32(0))

    @pl.when(wid == 0)
    def _():
        for q in range(BJ // 128):
            pltpu.sync_copy(bigv, perm_hbm.at[pl.ds(npad + q * 128, 128)])

    for i in range(NB):
        hist_s[i] = jnp.int32(0)

    def hloop(c, carry):
        v = buv[pl.ds(c * 16, 16)]
        for l in range(16):
            b = v[l]
            hist_s[b] = hist_s[b] + 1
        return carry

    lax.fori_loop(0, m // 16, hloop, jnp.int32(0))

    for c in range(NB // 16):
        v = jnp.zeros((16,), jnp.int32)
        for l in range(16):
            v = jnp.where(lanes == l, hist_s[c * 16 + l], v)
        histv[pl.ds(c * 16, 16)] = v

    pltpu.sync_copy(histv, hists_hbm.at[wid])
    plsc.subcore_barrier()
    pltpu.sync_copy(hists_hbm, allh)

    carry = jnp.int32(0)
    for c in range(NB // 16):
        sl = pl.ds(c * 16, 16)
        tot = jnp.zeros((16,), jnp.int32)
        for t in range(NT):
            tot = tot + allh[t, sl]
        excl = jnp.zeros((16,), jnp.int32)
        for l in range(16):
            excl = jnp.where(lanes == l, carry, excl)
            carry = carry + tot[l]

        def mloop(t, mb):
            return mb + allh[t, sl]

        mybase = lax.fori_loop(0, wid, mloop, jnp.zeros((16,), jnp.int32))
        bvec = excl + mybase
        basev[sl] = bvec
        for l in range(16):
            cnt_s[c * 16 + l] = bvec[l]

    @pl.when(wid == 0)
    def _():
        pltpu.sync_copy(basev, bstart_hbm)

    def ploop(c, carry):
        v = buv[pl.ds(c * 16, 16)]
        pv = jnp.zeros((16,), jnp.int32)
        for l in range(16):
            b = v[l]
            p = cnt_s[b]
            cnt_s[b] = p + 1
            pv = jnp.where(lanes == l, p, pv)
        posv[pl.ds(c * 16, 16)] = pv
        return carry

    lax.fori_loop(0, m // 16, ploop, jnp.int32(0))

    pltpu.async_copy(rowsv, perm_hbm.at[posv], sem).wait()


def _bucket_permute(pk, bu):
    npad = bu.shape[0]
    m = npad // NT
    fv = jnp.float32
    iv = jnp.int32
    mesh = plsc.VectorSubcoreMesh(core_axis_name="c", subcore_axis_name="s",
                                  num_cores=1)
    f = pl.kernel(
        _bucket_body,
        mesh=mesh,
        out_type=[
            jax.ShapeDtypeStruct((npad + BJ, PW), fv),
            jax.ShapeDtypeStruct((NB,), iv),
            jax.ShapeDtypeStruct((NT, NB), iv),
        ],
        scratch_types=[
            pltpu.VMEM((m, PW), fv),
            pltpu.VMEM((m,), iv),
            pltpu.VMEM((m,), iv),
            pltpu.VMEM((NB,), iv),
            pltpu.VMEM((NT, NB), iv),
            pltpu.VMEM((NB,), iv),
            pltpu.VMEM((128, PW), fv),
            pltpu.SMEM((NB,), iv),
            pltpu.SMEM((NB,), iv),
            pltpu.SemaphoreType.DMA,
        ],
    )
    return f(pk, bu)


def _nms_decay_body(c0_ref, c1_ref, pk_ref, out_ref):
    b = pl.program_id(0)
    lo = c0_ref[b]
    nch = c1_ref[b]
    bb = pl.multiple_of(b * BI, BI)

    blki = pk_ref[pl.ds(bb, BI), :]
    x1i = blki[:, 0:1]
    y1i = blki[:, 1:2]
    x2i = blki[:, 2:3]
    y2i = blki[:, 3:4]
    si = blki[:, 4:5]
    area_i = (x2i - x1i) * (y2i - y1i)

    def body(c, acc):
        st = pl.ds(pl.multiple_of(lo + c * BJ, 128), BJ)
        blkj = pk_ref[st, :]
        x1j = blkj[:, 0:1].reshape(1, BJ)
        y1j = blkj[:, 1:2].reshape(1, BJ)
        x2j = blkj[:, 2:3].reshape(1, BJ)
        y2j = blkj[:, 3:4].reshape(1, BJ)
        sj = blkj[:, 4:5].reshape(1, BJ)
        area_j = (x2j - x1j) * (y2j - y1j)

        w = jnp.maximum(jnp.minimum(x2i, x2j) - jnp.maximum(x1i, x1j), 0.0)
        h = jnp.maximum(jnp.minimum(y2i, y2j) - jnp.maximum(y1i, y1j), 0.0)
        inter = w * h
        union = (area_i + area_j) - inter
        iou = inter / union
        cond = jnp.logical_and(iou > IOU_THR, sj > si)
        f = jnp.where(cond, 1.0 - iou, 1.0)
        return acc * f

    acc = jax.lax.fori_loop(0, nch, body,
                            jnp.ones((BI, BJ), jnp.float32))

    width = BJ
    while width > 1:
        width //= 2
        acc = acc[:, :width] * acc[:, width:2 * width]

    out_ref[...] = si * acc


@jax.jit
def kernel(boxes, scores):
    n = boxes.shape[0]
    npad = ((n + BI - 1) // BI) * BI
    pad = npad - n

    x1 = boxes[:, 0]
    y1 = boxes[:, 1]
    x2p = boxes[:, 2] + 1.0
    y2p = boxes[:, 3] + 1.0
    maxext = jnp.maximum(jnp.max(x2p - x1), jnp.max(y2p - y1))
    minx = jnp.min(x1)
    cell = jnp.maximum(maxext, (jnp.max(x1) - minx) / NB)

    fullc = lambda a, v: jnp.pad(a, (0, pad), constant_values=v)
    x1f = fullc(x1, BIG)
    bu = jnp.clip((x1f - minx) / cell, 0.0, NB - 1).astype(jnp.int32)

    idxf = jnp.arange(npad, dtype=jnp.float32)
    pk6 = jnp.stack([x1f, fullc(y1, BIG), fullc(x2p, BIG),
                     fullc(y2p, BIG), fullc(scores, -BIG), idxf], axis=1)
    pk = jnp.pad(pk6, ((0, 0), (0, PW - 6)))

    perm, bstart, _ = _bucket_permute(pk, bu)
    order = perm[:npad, 5].astype(jnp.int32)

    nb = npad // BI
    bstart_ext = jnp.concatenate([bstart, jnp.array([npad], jnp.int32)])
    r0 = jnp.arange(nb, dtype=jnp.int32) * BI
    r1 = r0 + BI - 1
    bu0 = jnp.searchsorted(bstart, r0, side='right').astype(jnp.int32) - 1
    bu1 = jnp.searchsorted(bstart, r1, side='right').astype(jnp.int32) - 1
    lo_idx = bstart_ext[jnp.clip(bu0 - 1, 0, NB)]
    hi_idx = bstart_ext[jnp.clip(bu1 + 2, 0, NB)]
    lo_row = (lo_idx // 128 * 128).astype(jnp.int32)
    c0 = lo_row
    c1 = ((hi_idx - lo_row + BJ - 1) // BJ).astype(jnp.int32)

    sspec = pl.BlockSpec(memory_space=pltpu.SMEM)

    out = pl.pallas_call(
        _nms_decay_body,
        grid=(nb,),
        in_specs=[sspec, sspec,
                  pl.BlockSpec((npad + BJ, PW), lambda i: (0, 0))],
        out_specs=pl.BlockSpec((BI, 1), lambda i: (i, 0)),
        out_shape=jax.ShapeDtypeStruct((npad, 1), jnp.float32),
    )(c0, c1, perm)

    return jnp.zeros((npad,), jnp.float32).at[order].set(out[:, 0])[:n]


# speedup vs baseline: 18.2180x; 18.2180x over previous
import functools

import jax
import jax.numpy as jnp
from jax.experimental import pallas as pl
from jax.experimental.pallas import tpu as pltpu

IOU_THR = 0.4
BI = 512
BJ = 512
BIG = 1e30


def _nms_decay_body(c0_ref, c1_ref,
                    x1i_ref, y1i_ref, x2i_ref, y2i_ref, si_ref,
                    x1j_ref, y1j_ref, x2j_ref, y2j_ref, sj_ref,
                    out_ref):
    b = pl.program_id(0)
    lo = c0_ref[b]
    nch = c1_ref[b]

    x1i = x1i_ref[...]
    y1i = y1i_ref[...]
    x2i = x2i_ref[...]
    y2i = y2i_ref[...]
    si = si_ref[...]
    area_i = (x2i - x1i) * (y2i - y1i)

    def body(c, acc):
        sl = pl.ds(pl.multiple_of(lo + c * BJ, 128), BJ)
        x1j = x1j_ref[:, sl]
        y1j = y1j_ref[:, sl]
        x2j = x2j_ref[:, sl]
        y2j = y2j_ref[:, sl]
        sj = sj_ref[:, sl]
        area_j = (x2j - x1j) * (y2j - y1j)

        w = jnp.maximum(jnp.minimum(x2i, x2j) - jnp.maximum(x1i, x1j), 0.0)
        h = jnp.maximum(jnp.minimum(y2i, y2j) - jnp.maximum(y1i, y1j), 0.0)
        inter = w * h
        union = (area_i + area_j) - inter
        iou = inter / union
        cond = jnp.logical_and(iou > IOU_THR, sj > si)
        f = jnp.where(cond, 1.0 - iou, 1.0)
        return acc * f

    acc = jax.lax.fori_loop(0, nch, body,
                            jnp.ones((BI, BJ), jnp.float32))

    width = BJ
    while width > 1:
        width //= 2
        acc = acc[:, :width] * acc[:, width:2 * width]

    out_ref[...] = si * acc


@jax.jit
def kernel(boxes, scores):
    n = boxes.shape[0]
    npad = ((n + BI - 1) // BI) * BI
    pad = npad - n

    x1 = boxes[:, 0]
    y1 = boxes[:, 1]
    x2p = boxes[:, 2] + 1.0
    y2p = boxes[:, 3] + 1.0
    maxext = jnp.maximum(jnp.max(x2p - x1), jnp.max(y2p - y1))

    iota = jnp.arange(n, dtype=jnp.int32)
    xs1, ys1, xs2, ys2, ss, order = jax.lax.sort(
        (x1, y1, x2p, y2p, scores, iota), num_keys=1)
    xs1 = jnp.pad(xs1, (0, pad + BJ), constant_values=BIG)
    ys1 = jnp.pad(ys1, (0, pad + BJ), constant_values=BIG)
    xs2 = jnp.pad(xs2, (0, pad + BJ), constant_values=BIG)
    ys2 = jnp.pad(ys2, (0, pad + BJ), constant_values=BIG)
    ss = jnp.pad(ss, (0, pad + BJ), constant_values=-BIG)

    nb = npad // BI
    blk = xs1[:npad].reshape(nb, BI)
    lo_idx = jnp.searchsorted(xs1[:npad], blk[:, 0] - maxext, side='left')
    hi_idx = jnp.searchsorted(xs1[:npad], blk[:, -1] + maxext, side='right')
    lo_row = (lo_idx // 128 * 128).astype(jnp.int32)
    c0 = lo_row
    c1 = ((hi_idx - lo_row + BJ - 1) // BJ).astype(jnp.int32)

    col = lambda a: a[:npad].reshape(npad, 1)
    row = lambda a: a.reshape(1, npad + BJ)

    ispec = pl.BlockSpec((BI, 1), lambda i: (i, 0))
    jspec = pl.BlockSpec((1, npad + BJ), lambda i: (0, 0))
    sspec = pl.BlockSpec(memory_space=pltpu.SMEM)

    out = pl.pallas_call(
        _nms_decay_body,
        grid=(nb,),
        in_specs=[sspec, sspec,
                  ispec, ispec, ispec, ispec, ispec,
                  jspec, jspec, jspec, jspec, jspec],
        out_specs=pl.BlockSpec((BI, 1), lambda i: (i, 0)),
        out_shape=jax.ShapeDtypeStruct((npad, 1), jnp.float32),
    )(c0, c1,
      col(xs1), col(ys1), col(xs2), col(ys2), col(ss),
      row(xs1), row(ys1), row(xs2), row(ys2), row(ss))

    decayed_sorted = out[:n, 0]
    return jnp.zeros((n,), jnp.float32).at[order].set(decayed_sorted)
